# Initial kernel scaffold; baseline (speedup 1.0000x reference)
import jax
import jax.numpy as jnp
from jax.experimental import pallas as pl


def kernel(x_lnc, x_dis, ei0, ei1, ei2, ei3, pos_edges, neg_edges, Wl0, Wl1, bng, bnb, Ws0, bs0, Ws1, bs1, gW0, gal0, gar0, gb0, gW1, gal1, gar1, gb1, gW2, gal2, gar2, gb2, gW3, gal3, gar3, gb3, saW1, sab1, saW2):
    raise NotImplementedError("write your pallas kernel here")


# speedup vs baseline: 7.4477x; 7.4477x over previous
import jax
import jax.numpy as jnp
from jax.experimental import pallas as pl

N_NODE = 25000
HID = 128
FOUT = 256
HEADS = 4
OUT = 64
BLK = 5000
NBLK = N_NODE // BLK
E_POS = 100000
EBLK = 4000


def _leaky(x, s):
    return jnp.where(x >= 0, x, s * x)


def _proj_stats_kernel(x_ref, wl_ref, ws_ref, bs_ref, y_ref, skip_ref, st_ref):
    i = pl.program_id(0)
    x = x_ref[...]
    y = jnp.dot(x, wl_ref[...].T, preferred_element_type=jnp.float32)
    y_ref[...] = y
    skip_ref[...] = jnp.dot(x, ws_ref[...].T, preferred_element_type=jnp.float32) + bs_ref[...]

    @pl.when(i == 0)
    def _():
        st_ref[...] = jnp.zeros_like(st_ref)

    s = jnp.sum(y, axis=0, keepdims=True)
    q = jnp.sum(y * y, axis=0, keepdims=True)
    st_ref[...] += jnp.concatenate([s, q, jnp.zeros((6, HID), jnp.float32)], axis=0)


def _proj_stats(x, wl, ws, bs):
    return pl.pallas_call(
        _proj_stats_kernel,
        grid=(NBLK,),
        in_specs=[
            pl.BlockSpec((BLK, HID), lambda i: (i, 0)),
            pl.BlockSpec((HID, HID), lambda i: (0, 0)),
            pl.BlockSpec((HID, HID), lambda i: (0, 0)),
            pl.BlockSpec((1, HID), lambda i: (0, 0)),
        ],
        out_specs=[
            pl.BlockSpec((BLK, HID), lambda i: (i, 0)),
            pl.BlockSpec((BLK, HID), lambda i: (i, 0)),
            pl.BlockSpec((8, HID), lambda i: (0, 0)),
        ],
        out_shape=[
            jax.ShapeDtypeStruct((N_NODE, HID), jnp.float32),
            jax.ShapeDtypeStruct((N_NODE, HID), jnp.float32),
            jax.ShapeDtypeStruct((8, HID), jnp.float32),
        ],
    )(x, wl, ws, bs)


def _proj_apply_kernel(y_ref, skip_ref, st_ref, g_ref, b_ref, o_ref):
    st = st_ref[...]
    mu = st[0:1, :] / N_NODE
    var = st[1:2, :] / N_NODE - mu * mu
    ybn = (y_ref[...] - mu) * jax.lax.rsqrt(var + 1e-5) * g_ref[...] + b_ref[...]
    o_ref[...] = _leaky(ybn, 0.01) + skip_ref[...]


def _proj_apply(y, skip, st, g, b):
    return pl.pallas_call(
        _proj_apply_kernel,
        grid=(NBLK,),
        in_specs=[
            pl.BlockSpec((BLK, HID), lambda i: (i, 0)),
            pl.BlockSpec((BLK, HID), lambda i: (i, 0)),
            pl.BlockSpec((8, HID), lambda i: (0, 0)),
            pl.BlockSpec((1, HID), lambda i: (0, 0)),
            pl.BlockSpec((1, HID), lambda i: (0, 0)),
        ],
        out_specs=pl.BlockSpec((BLK, HID), lambda i: (i, 0)),
        out_shape=jax.ShapeDtypeStruct((N_NODE, HID), jnp.float32),
    )(y, skip, st, g, b)


def _gat_src_kernel(h_ref, w_ref, aal_ref, aar_ref, fs_ref, el_ref, er_ref):
    fs = jnp.dot(h_ref[...], w_ref[...].T, preferred_element_type=jnp.float32)
    fs_ref[...] = fs
    el_ref[...] = jnp.dot(fs, aal_ref[...], preferred_element_type=jnp.float32)
    er_ref[...] = jnp.dot(fs, aar_ref[...], preferred_element_type=jnp.float32)


def _gat_src(h, w, aal, aar):
    return pl.pallas_call(
        _gat_src_kernel,
        grid=(NBLK,),
        in_specs=[
            pl.BlockSpec((BLK, HID), lambda i: (i, 0)),
            pl.BlockSpec((FOUT, HID), lambda i: (0, 0)),
            pl.BlockSpec((FOUT, HEADS), lambda i: (0, 0)),
            pl.BlockSpec((FOUT, HEADS), lambda i: (0, 0)),
        ],
        out_specs=[
            pl.BlockSpec((BLK, FOUT), lambda i: (i, 0)),
            pl.BlockSpec((BLK, HEADS), lambda i: (i, 0)),
            pl.BlockSpec((BLK, HEADS), lambda i: (i, 0)),
        ],
        out_shape=[
            jax.ShapeDtypeStruct((N_NODE, FOUT), jnp.float32),
            jax.ShapeDtypeStruct((N_NODE, HEADS), jnp.float32),
            jax.ShapeDtypeStruct((N_NODE, HEADS), jnp.float32),
        ],
    )(h, w, aal, aar)


def _gat_dst_kernel(h_ref, w_ref, aar_ref, er_ref):
    fd = jnp.dot(h_ref[...], w_ref[...].T, preferred_element_type=jnp.float32)
    er_ref[...] = jnp.dot(fd, aar_ref[...], preferred_element_type=jnp.float32)


def _gat_dst(h, w, aar):
    return pl.pallas_call(
        _gat_dst_kernel,
        grid=(NBLK,),
        in_specs=[
            pl.BlockSpec((BLK, HID), lambda i: (i, 0)),
            pl.BlockSpec((FOUT, HID), lambda i: (0, 0)),
            pl.BlockSpec((FOUT, HEADS), lambda i: (0, 0)),
        ],
        out_specs=pl.BlockSpec((BLK, HEADS), lambda i: (i, 0)),
        out_shape=jax.ShapeDtypeStruct((N_NODE, HEADS), jnp.float32),
    )(h, w, aar)


def _sem_score_kernel(ra_ref, rb_ref, ba_ref, bb_ref, w1_ref, b1_ref,
                      ea_ref, eb_ref, csa_ref, csb_ref):
    ea = _leaky(ra_ref[...] + ba_ref[...], 0.01)
    eb = _leaky(rb_ref[...] + bb_ref[...], 0.01)
    ea_ref[...] = ea
    eb_ref[...] = eb
    ta = jnp.tanh(jnp.dot(ea, w1_ref[...].T, preferred_element_type=jnp.float32) + b1_ref[...])
    tb = jnp.tanh(jnp.dot(eb, w1_ref[...].T, preferred_element_type=jnp.float32) + b1_ref[...])
    csa_ref[...] = jnp.sum(ta, axis=0, keepdims=True)[None]
    csb_ref[...] = jnp.sum(tb, axis=0, keepdims=True)[None]


def _sem_score(ra, rb, ba, bb, w1, b1):
    return pl.pallas_call(
        _sem_score_kernel,
        grid=(NBLK,),
        in_specs=[
            pl.BlockSpec((BLK, FOUT), lambda i: (i, 0)),
            pl.BlockSpec((BLK, FOUT), lambda i: (i, 0)),
            pl.BlockSpec((1, FOUT), lambda i: (0, 0)),
            pl.BlockSpec((1, FOUT), lambda i: (0, 0)),
            pl.BlockSpec((HID, FOUT), lambda i: (0, 0)),
            pl.BlockSpec((1, HID), lambda i: (0, 0)),
        ],
        out_specs=[
            pl.BlockSpec((BLK, FOUT), lambda i: (i, 0)),
            pl.BlockSpec((BLK, FOUT), lambda i: (i, 0)),
            pl.BlockSpec((1, 1, HID), lambda i: (i, 0, 0)),
            pl.BlockSpec((1, 1, HID), lambda i: (i, 0, 0)),
        ],
        out_shape=[
            jax.ShapeDtypeStruct((N_NODE, FOUT), jnp.float32),
            jax.ShapeDtypeStruct((N_NODE, FOUT), jnp.float32),
            jax.ShapeDtypeStruct((NBLK, 1, HID), jnp.float32),
            jax.ShapeDtypeStruct((NBLK, 1, HID), jnp.float32),
        ],
    )(ra, rb, ba, bb, w1, b1)


def _sem_combine_kernel(ea_ref, eb_ref, bta_ref, btb_ref, o_ref):
    o_ref[...] = bta_ref[...] * ea_ref[...] + btb_ref[...] * eb_ref[...]


def _sem_combine(ea, eb, bta, btb):
    return pl.pallas_call(
        _sem_combine_kernel,
        grid=(NBLK,),
        in_specs=[
            pl.BlockSpec((BLK, FOUT), lambda i: (i, 0)),
            pl.BlockSpec((BLK, FOUT), lambda i: (i, 0)),
            pl.BlockSpec((1, FOUT), lambda i: (0, 0)),
            pl.BlockSpec((1, FOUT), lambda i: (0, 0)),
        ],
        out_specs=pl.BlockSpec((BLK, FOUT), lambda i: (i, 0)),
        out_shape=jax.ShapeDtypeStruct((N_NODE, FOUT), jnp.float32),
    )(ea, eb, bta, btb)


def _mul_kernel(a_ref, b_ref, o_ref):
    o_ref[...] = a_ref[...] * b_ref[...]


def _pmul(a, b):
    nb = E_POS // EBLK
    return pl.pallas_call(
        _mul_kernel,
        grid=(nb,),
        in_specs=[
            pl.BlockSpec((EBLK, FOUT), lambda i: (i, 0)),
            pl.BlockSpec((EBLK, FOUT), lambda i: (i, 0)),
        ],
        out_specs=pl.BlockSpec((EBLK, FOUT), lambda i: (i, 0)),
        out_shape=jax.ShapeDtypeStruct((E_POS, FOUT), jnp.float32),
    )(a, b)


def _edge_softmax_aggregate(fs, el, er, edges):
    src = edges[0]
    dst = edges[1]
    e = _leaky(el[src] + er[dst], 0.2)
    m = jax.ops.segment_max(e, dst, num_segments=N_NODE)
    m = jnp.where(jnp.isfinite(m), m, 0.0)
    ex = jnp.exp(e - m[dst])
    den = jax.ops.segment_sum(ex, dst, num_segments=N_NODE)
    alpha = ex / den[dst]
    w = jnp.repeat(alpha, OUT, axis=1)
    return jax.ops.segment_sum(fs[src] * w, dst, num_segments=N_NODE)


def _blockdiag(a):
    return (a[:, :, None] * jnp.eye(HEADS, dtype=a.dtype)[:, None, :]).reshape(FOUT, HEADS)


def kernel(x_lnc, x_dis, ei0, ei1, ei2, ei3, pos_edges, neg_edges, Wl0, Wl1, bng, bnb, Ws0, bs0, Ws1, bs1, gW0, gal0, gar0, gb0, gW1, gal1, gar1, gb1, gW2, gal2, gar2, gb2, gW3, gal3, gar3, gb3, saW1, sab1, saW2):
    bng2 = bng.reshape(1, HID)
    bnb2 = bnb.reshape(1, HID)
    yl, skl, stl = _proj_stats(x_lnc, Wl0, Ws0, bs0.reshape(1, HID))
    yd, skd, std_ = _proj_stats(x_dis, Wl1, Ws1, bs1.reshape(1, HID))
    l = _proj_apply(yl, skl, stl, bng2, bnb2)
    d = _proj_apply(yd, skd, std_, bng2, bnb2)

    rsts = []
    for (hs, hd, ei, gW, gal, gar) in (
        (l, d, ei0, gW0, gal0, gar0),
        (l, l, ei1, gW1, gal1, gar1),
        (d, l, ei2, gW2, gal2, gar2),
        (d, d, ei3, gW3, gal3, gar3),
    ):
        aal = _blockdiag(gal)
        aar = _blockdiag(gar)
        fs, el, er_s = _gat_src(hs, gW, aal, aar)
        if hs is hd:
            er = er_s
        else:
            er = _gat_dst(hd, gW, aar)
        rsts.append(_edge_softmax_aggregate(fs, el, er, ei))

    sab1_2 = sab1.reshape(1, HID)

    def semattn(ra, rb, ba, bb):
        ea, eb, csa, csb = _sem_score(ra, rb, ba.reshape(1, FOUT),
                                      bb.reshape(1, FOUT), saW1, sab1_2)
        wa = jnp.sum(csa.reshape(NBLK, HID), axis=0) @ saW2[0] / N_NODE
        wb = jnp.sum(csb.reshape(NBLK, HID), axis=0) @ saW2[0] / N_NODE
        beta = jax.nn.softmax(jnp.stack([wa, wb]))
        bta = jnp.full((1, FOUT), beta[0], jnp.float32)
        btb = jnp.full((1, FOUT), beta[1], jnp.float32)
        return _sem_combine(ea, eb, bta, btb)

    l_embs = semattn(rsts[1], rsts[2], gb1, gb2)
    d_embs = semattn(rsts[0], rsts[3], gb0, gb3)

    pos_z = _pmul(jnp.take(l_embs, pos_edges[0], axis=0),
                  jnp.take(d_embs, pos_edges[1], axis=0))
    neg_z = _pmul(jnp.take(l_embs, neg_edges[0], axis=0),
                  jnp.take(d_embs, neg_edges[1], axis=0))
    return (pos_z, neg_z, l_embs, d_embs)
